# Initial kernel scaffold; baseline (speedup 1.0000x reference)
import jax
import jax.numpy as jnp
from jax.experimental import pallas as pl


def kernel(f_nuc, f_bond, node_graph, message_graph, all_bonds, scope, W_local, W_msg, W_node_emb, gru_w_ih, gru_w_hh, gru_b_ih, gru_b_hh, lstm_w_ih_f, lstm_w_hh_f, lstm_b_ih_f, lstm_b_hh_f, lstm_w_ih_b, lstm_w_hh_b, lstm_b_ih_b, lstm_b_hh_b):
    raise NotImplementedError("write your pallas kernel here")


# speedup vs baseline: 2.6566x; 2.6566x over previous
import functools

import jax
import jax.numpy as jnp
from jax import lax
from jax.experimental import pallas as pl
from jax.experimental.pallas import tpu as pltpu
from jax.experimental.pallas import tpu_sc as plsc

F32 = jnp.float32


def _sc_info():
    try:
        info = plsc.get_sparse_core_info()
        return info.num_cores, info.num_subcores
    except Exception:
        return 2, 16


def _build_gather_sum(P, n_rows, n_chunks, C, NC, NS):
    mesh = plsc.VectorSubcoreMesh(core_axis_name="c", subcore_axis_name="s")

    def body(msgs_hbm, i0_hbm, i1_hbm, i2_hbm, out_hbm, i0_v, i1_v, i2_v, rows_v, sem):
        wid = lax.axis_index("s") * NC + lax.axis_index("c")
        base0 = wid * n_rows

        def chunk(k, carry):
            base = base0 + k * C
            pltpu.sync_copy(i0_hbm.at[pl.ds(base, C)], i0_v)
            pltpu.sync_copy(i1_hbm.at[pl.ds(base, C)], i1_v)
            pltpu.sync_copy(i2_hbm.at[pl.ds(base, C)], i2_v)
            d0 = pltpu.async_copy(msgs_hbm.at[i0_v], rows_v.at[0], sem)
            d1 = pltpu.async_copy(msgs_hbm.at[i1_v], rows_v.at[1], sem)
            d2 = pltpu.async_copy(msgs_hbm.at[i2_v], rows_v.at[2], sem)
            d0.wait()
            d1.wait()
            d2.wait()

            def row(r, c2):
                for l in range(8):
                    sl = pl.ds(l * 16, 16)
                    rows_v[0, r, sl] = rows_v[0, r, sl] + rows_v[1, r, sl] + rows_v[2, r, sl]
                return c2

            lax.fori_loop(0, C, row, 0)
            pltpu.sync_copy(rows_v.at[0], out_hbm.at[pl.ds(base, C)])
            return carry

        lax.fori_loop(0, n_chunks, chunk, 0)

    return pl.kernel(
        body,
        out_type=jax.ShapeDtypeStruct((P, 128), F32),
        mesh=mesh,
        scratch_types=[
            pltpu.VMEM((C,), jnp.int32),
            pltpu.VMEM((C,), jnp.int32),
            pltpu.VMEM((C,), jnp.int32),
            pltpu.VMEM((3, C, 128), F32),
            pltpu.SemaphoreType.DMA,
        ],
    )


def _local_potentials(f_bond_p, WlT, BM):
    EP, K = f_bond_p.shape
    H = WlT.shape[1]

    def body(fb_ref, w_ref, lp_ref, msg_ref):
        lp = jnp.dot(fb_ref[...], w_ref[...], preferred_element_type=F32)
        lp_ref[...] = lp
        msg_ref[...] = jnp.maximum(lp, 0.0)

    return pl.pallas_call(
        body,
        grid=(EP // BM,),
        in_specs=[
            pl.BlockSpec((BM, K), lambda i: (i, 0)),
            pl.BlockSpec((K, H), lambda i: (0, 0)),
        ],
        out_specs=[
            pl.BlockSpec((BM, H), lambda i: (i, 0)),
            pl.BlockSpec((BM, H), lambda i: (i, 0)),
        ],
        out_shape=[
            jax.ShapeDtypeStruct((EP, H), F32),
            jax.ShapeDtypeStruct((EP, H), F32),
        ],
    )(f_bond_p, WlT)


def _gru_update(sn, lp, msg, WmT, WihT, WhhT, bih, bhh, BM):
    EP, H = sn.shape

    def body(sn_ref, lp_ref, msg_ref, wm_ref, wi_ref, wh_ref, bi_ref, bh_ref, out_ref):
        nb = jnp.dot(sn_ref[...], wm_ref[...], preferred_element_type=F32)
        new = jnp.maximum(lp_ref[...] + nb, 0.0)
        h = msg_ref[...]
        gi = jnp.dot(new, wi_ref[...], preferred_element_type=F32) + bi_ref[...]
        gh = jnp.dot(h, wh_ref[...], preferred_element_type=F32) + bh_ref[...]
        r = jax.nn.sigmoid(gi[:, :H] + gh[:, :H])
        z = jax.nn.sigmoid(gi[:, H:2 * H] + gh[:, H:2 * H])
        n = jnp.tanh(gi[:, 2 * H:] + r * gh[:, 2 * H:])
        out_ref[...] = (1.0 - z) * n + z * h

        @pl.when(pl.program_id(0) == 0)
        def _():
            out_ref[0:1, :] = jnp.zeros((1, H), F32)

    return pl.pallas_call(
        body,
        grid=(EP // BM,),
        in_specs=[
            pl.BlockSpec((BM, H), lambda i: (i, 0)),
            pl.BlockSpec((BM, H), lambda i: (i, 0)),
            pl.BlockSpec((BM, H), lambda i: (i, 0)),
            pl.BlockSpec((H, H), lambda i: (0, 0)),
            pl.BlockSpec((H, 3 * H), lambda i: (0, 0)),
            pl.BlockSpec((H, 3 * H), lambda i: (0, 0)),
            pl.BlockSpec((1, 3 * H), lambda i: (0, 0)),
            pl.BlockSpec((1, 3 * H), lambda i: (0, 0)),
        ],
        out_specs=pl.BlockSpec((BM, H), lambda i: (i, 0)),
        out_shape=jax.ShapeDtypeStruct((EP, H), F32),
    )(sn, lp, msg, WmT, WihT, WhhT, bih, bhh)


def _node_embedding(f_nuc_p, nnm, W1T, W2T, BM):
    NP, K = f_nuc_p.shape
    H = W2T.shape[1]

    def body(fn_ref, nm_ref, w1_ref, w2_ref, out_ref):
        acc = jnp.dot(fn_ref[...], w1_ref[...], preferred_element_type=F32)
        acc = acc + jnp.dot(nm_ref[...], w2_ref[...], preferred_element_type=F32)
        out_ref[...] = jnp.maximum(acc, 0.0)

    return pl.pallas_call(
        body,
        grid=(NP // BM,),
        in_specs=[
            pl.BlockSpec((BM, K), lambda i: (i, 0)),
            pl.BlockSpec((BM, H), lambda i: (i, 0)),
            pl.BlockSpec((K, H), lambda i: (0, 0)),
            pl.BlockSpec((H, H), lambda i: (0, 0)),
        ],
        out_specs=pl.BlockSpec((BM, H), lambda i: (i, 0)),
        out_shape=jax.ShapeDtypeStruct((NP, H), F32),
    )(f_nuc_p, nnm, W1T, W2T)


def _bilstm_maxpool(ne_t, WifT, WhfT, bf, WibT, WhbT, bb, T_b):
    Lq, Bq, H = ne_t.shape
    HH = WhfT.shape[0]
    G = Lq // T_b

    def body(nef_ref, neb_ref, wif_ref, whf_ref, bf_ref, wib_ref, whb_ref, bb_ref,
             out_ref, hf_s, cf_s, hb_s, cb_s, mf_s, mb_s):
        i = pl.program_id(0)

        @pl.when(i == 0)
        def _():
            z = jnp.zeros((Bq, HH), F32)
            hf_s[...] = z
            cf_s[...] = z
            hb_s[...] = z
            cb_s[...] = z
            m0 = jnp.full((Bq, HH), -jnp.inf, F32)
            mf_s[...] = m0
            mb_s[...] = m0

        def one_dir(x, h, c, wi_ref, wh_ref, b_ref):
            g = (jnp.dot(x, wi_ref[...], preferred_element_type=F32)
                 + jnp.dot(h, wh_ref[...], preferred_element_type=F32)
                 + b_ref[...])
            ig = jax.nn.sigmoid(g[:, :HH])
            fg = jax.nn.sigmoid(g[:, HH:2 * HH])
            gg = jnp.tanh(g[:, 2 * HH:3 * HH])
            og = jax.nn.sigmoid(g[:, 3 * HH:])
            c = fg * c + ig * gg
            h = og * jnp.tanh(c)
            return h, c

        def step(tt, carry):
            hf, cf, hb, cb, mf, mb = carry
            hf, cf = one_dir(nef_ref[tt], hf, cf, wif_ref, whf_ref, bf_ref)
            mf = jnp.maximum(mf, hf)
            hb, cb = one_dir(neb_ref[T_b - 1 - tt], hb, cb, wib_ref, whb_ref, bb_ref)
            mb = jnp.maximum(mb, hb)
            return hf, cf, hb, cb, mf, mb

        init = (hf_s[...], cf_s[...], hb_s[...], cb_s[...], mf_s[...], mb_s[...])
        hf, cf, hb, cb, mf, mb = lax.fori_loop(0, T_b, step, init)
        hf_s[...] = hf
        cf_s[...] = cf
        hb_s[...] = hb
        cb_s[...] = cb
        mf_s[...] = mf
        mb_s[...] = mb

        @pl.when(i == G - 1)
        def _():
            out_ref[...] = jnp.concatenate([mf, mb], axis=1)

    return pl.pallas_call(
        body,
        grid=(G,),
        in_specs=[
            pl.BlockSpec((T_b, Bq, H), lambda i: (i, 0, 0)),
            pl.BlockSpec((T_b, Bq, H), lambda i: (G - 1 - i, 0, 0)),
            pl.BlockSpec((H, 4 * HH), lambda i: (0, 0)),
            pl.BlockSpec((HH, 4 * HH), lambda i: (0, 0)),
            pl.BlockSpec((1, 4 * HH), lambda i: (0, 0)),
            pl.BlockSpec((H, 4 * HH), lambda i: (0, 0)),
            pl.BlockSpec((HH, 4 * HH), lambda i: (0, 0)),
            pl.BlockSpec((1, 4 * HH), lambda i: (0, 0)),
        ],
        out_specs=pl.BlockSpec((Bq, 2 * HH), lambda i: (0, 0)),
        out_shape=jax.ShapeDtypeStruct((Bq, 2 * HH), F32),
        scratch_shapes=[pltpu.VMEM((Bq, HH), F32)] * 6,
    )(ne_t, ne_t, WifT, WhfT, bf, WibT, WhbT, bb)


def _pad_rows(x, P):
    n = x.shape[0]
    if n == P:
        return x
    return jnp.concatenate(
        [x, jnp.zeros((P - n,) + x.shape[1:], x.dtype)], axis=0)


def kernel(f_nuc, f_bond, node_graph, message_graph, all_bonds, scope,
           W_local, W_msg, W_node_emb,
           gru_w_ih, gru_w_hh, gru_b_ih, gru_b_hh,
           lstm_w_ih_f, lstm_w_hh_f, lstm_b_ih_f, lstm_b_hh_f,
           lstm_w_ih_b, lstm_w_hh_b, lstm_b_ih_b, lstm_b_hh_b):
    E = f_bond.shape[0]
    N = f_nuc.shape[0]
    H = W_msg.shape[0]
    B = scope.shape[0]
    L = N // B
    HH = lstm_w_hh_f.shape[1]
    NC, NS = _sc_info()
    NW = NC * NS
    C = 128
    unit = NW * C

    EP = ((E + unit - 1) // unit) * unit
    NP = ((N + unit - 1) // unit) * unit
    e_rows = EP // NW
    n_rows = NP // NW

    f_bond_p = _pad_rows(f_bond.astype(F32), EP)
    f_nuc_p = _pad_rows(f_nuc.astype(F32), NP)
    mg = _pad_rows(message_graph.astype(jnp.int32), EP)
    ng = _pad_rows(node_graph.astype(jnp.int32), NP)
    i0, i1, i2 = mg[:, 0], mg[:, 1], mg[:, 2]
    n0, n1, n2 = ng[:, 0], ng[:, 1], ng[:, 2]

    WlT = W_local.T.astype(F32)
    WmT = W_msg.T
    WihT = gru_w_ih.T
    WhhT = gru_w_hh.T
    bih = gru_b_ih.reshape(1, 3 * H)
    bhh = gru_b_hh.reshape(1, 3 * H)
    W1T = W_node_emb[:, :4].T
    W2T = W_node_emb[:, 4:].T
    WifT = lstm_w_ih_f.T
    WhfT = lstm_w_hh_f.T
    bf = (lstm_b_ih_f + lstm_b_hh_f).reshape(1, 4 * HH)
    WibT = lstm_w_ih_b.T
    WhbT = lstm_w_hh_b.T
    bb = (lstm_b_ih_b + lstm_b_hh_b).reshape(1, 4 * HH)

    BM = 2048
    lp, msgs = _local_potentials(f_bond_p, WlT, BM)

    gsum_e = _build_gather_sum(EP, e_rows, e_rows // C, C, NC, NS)
    for _ in range(2):
        sn = gsum_e(msgs, i0, i1, i2)
        msgs = _gru_update(sn, lp, msgs, WmT, WihT, WhhT, bih, bhh, BM)

    gsum_n = _build_gather_sum(NP, n_rows, n_rows // C, C, NC, NS)
    nnm = gsum_n(msgs, n0, n1, n2)

    ne = _node_embedding(f_nuc_p, nnm, W1T, W2T, BM)
    ne_t = ne[:N].reshape(B, L, H).transpose(1, 0, 2)

    T_b = 1
    for d in range(min(25, L), 0, -1):
        if L % d == 0:
            T_b = d
            break
    rep = _bilstm_maxpool(ne_t, WifT, WhfT, bf, WibT, WhbT, bb, T_b)
    return rep
